# initial kernel scaffold (unmeasured)
import jax
import jax.numpy as jnp
from jax import lax
from jax.experimental import pallas as pl
from jax.experimental.pallas import tpu as pltpu


def kernel(
    x,
):
    def body(*refs):
        pass

    out_shape = jax.ShapeDtypeStruct(..., jnp.float32)
    return pl.pallas_call(body, out_shape=out_shape)(...)


# baseline (device time: 2192547 ns/iter reference)
import jax
import jax.numpy as jnp
from jax import lax
from jax.experimental import pallas as pl
from jax.experimental.pallas import tpu as pltpu


def kernel(x):
    m_per, n = x.shape
    x_bf = x.astype(jnp.bfloat16)

    def body(x_ref, out_ref, copy_sem, send_sem, recv_sem):
        my_x = lax.axis_index("x")
        my_y = lax.axis_index("y")
        my_z = lax.axis_index("z")
        peer = (1 - my_x, my_y, my_z)

        barrier_sem = pltpu.get_barrier_semaphore()
        pl.semaphore_signal(
            barrier_sem, inc=1, device_id=peer,
            device_id_type=pl.DeviceIdType.MESH,
        )
        pl.semaphore_wait(barrier_sem, 1)

        local = pltpu.make_async_copy(
            x_ref, out_ref.at[pl.ds(my_x * m_per, m_per), :], copy_sem
        )
        local.start()

        rdma = pltpu.make_async_remote_copy(
            src_ref=x_ref,
            dst_ref=out_ref.at[pl.ds(my_x * m_per, m_per), :],
            send_sem=send_sem,
            recv_sem=recv_sem,
            device_id=peer,
            device_id_type=pl.DeviceIdType.MESH,
        )
        rdma.start()

        local.wait()
        rdma.wait_send()
        rdma.wait_recv()

    return pl.pallas_call(
        body,
        out_shape=jax.ShapeDtypeStruct((2 * m_per, n), jnp.bfloat16),
        in_specs=[pl.BlockSpec(memory_space=pl.ANY)],
        out_specs=pl.BlockSpec(memory_space=pl.ANY),
        scratch_shapes=[
            pltpu.SemaphoreType.DMA,
            pltpu.SemaphoreType.DMA,
            pltpu.SemaphoreType.DMA,
        ],
        compiler_params=pltpu.CompilerParams(collective_id=0),
    )(x_bf)


# device time: 812454 ns/iter; 2.6987x vs baseline; 2.6987x over previous
import jax
import jax.numpy as jnp
from jax import lax
from jax.experimental import pallas as pl
from jax.experimental.pallas import tpu as pltpu

NCHUNKS = 16


def kernel(x):
    m_per, n = x.shape
    rows = m_per // NCHUNKS

    def body(x_ref, out_ref, vin, vbf, ld_sems, st_sem, send_sems, recv_sems):
        my_x = lax.axis_index("x")
        my_y = lax.axis_index("y")
        my_z = lax.axis_index("z")
        peer = (1 - my_x, my_y, my_z)

        barrier_sem = pltpu.get_barrier_semaphore()
        pl.semaphore_signal(
            barrier_sem, inc=1, device_id=peer,
            device_id_type=pl.DeviceIdType.MESH,
        )
        pl.semaphore_wait(barrier_sem, 1)

        def in_chunk(i):
            return x_ref.at[pl.ds(i * rows, rows), :]

        def out_chunk(i):
            return out_ref.at[pl.ds(my_x * m_per + i * rows, rows), :]

        def rdma_chunk(i, slot):
            return pltpu.make_async_remote_copy(
                src_ref=vbf.at[slot],
                dst_ref=out_chunk(i),
                send_sem=send_sems.at[i],
                recv_sem=recv_sems.at[i],
                device_id=peer,
                device_id_type=pl.DeviceIdType.MESH,
            )

        lds = [None] * NCHUNKS
        sts = [None] * NCHUNKS
        sends = [None] * NCHUNKS
        for i in range(2):
            lds[i] = pltpu.make_async_copy(in_chunk(i), vin.at[i], ld_sems.at[i])
            lds[i].start()
        for i in range(NCHUNKS):
            slot = i % 2
            lds[i].wait()
            if i >= 2:
                sends[i - 2].wait_send()
            vbf[slot] = vin[slot].astype(jnp.bfloat16)
            nxt = i + 2
            if nxt < NCHUNKS:
                lds[nxt] = pltpu.make_async_copy(
                    in_chunk(nxt), vin.at[slot], ld_sems.at[slot]
                )
                lds[nxt].start()
            sends[i] = rdma_chunk(i, slot)
            sends[i].start()
            sts[i] = pltpu.make_async_copy(vbf.at[slot], out_chunk(i), st_sem)
            sts[i].start()
            sts[i].wait()
        for i in (NCHUNKS - 2, NCHUNKS - 1):
            sends[i].wait_send()
        for i in range(NCHUNKS):
            rdma_chunk(i, 0).wait_recv()

    return pl.pallas_call(
        body,
        out_shape=jax.ShapeDtypeStruct((2 * m_per, n), jnp.bfloat16),
        in_specs=[pl.BlockSpec(memory_space=pl.ANY)],
        out_specs=pl.BlockSpec(memory_space=pl.ANY),
        scratch_shapes=[
            pltpu.VMEM((2, rows, n), jnp.float32),
            pltpu.VMEM((2, rows, n), jnp.bfloat16),
            pltpu.SemaphoreType.DMA((2,)),
            pltpu.SemaphoreType.DMA,
            pltpu.SemaphoreType.DMA((NCHUNKS,)),
            pltpu.SemaphoreType.DMA((NCHUNKS,)),
        ],
        compiler_params=pltpu.CompilerParams(collective_id=0),
    )(x)
